# Initial kernel scaffold; baseline (speedup 1.0000x reference)
import jax
import jax.numpy as jnp
from jax.experimental import pallas as pl


def kernel(attn_mtx, embedding_table):
    raise NotImplementedError("write your pallas kernel here")


# speedup vs baseline: 48.4461x; 48.4461x over previous
import jax
import jax.numpy as jnp
from jax.experimental import pallas as pl
from jax.experimental.pallas import tpu as pltpu

_MAX_SEQLEN = 4096
_SEG_W = 4096
_TQ = 128
_TK = 2048


def _rel_bias_kernel(seg_ref, attn_ref, out_ref, bias_ref, u_ref):
    i = pl.program_id(1)

    @pl.when(i == 0)
    def _build_u():
        seg = seg_ref[0, 0, :]
        v = jnp.broadcast_to(seg[None, :], (8, _SEG_W))
        row = jax.lax.broadcasted_iota(jnp.int32, (8, _SEG_W), 0)
        for t in range(3):
            n = 1 << t
            rolled = pltpu.roll(v, _SEG_W - n, axis=1)
            v = jnp.where((row & n) == 0, rolled, v)
        for a in range(16):
            n = 120 - 8 * a
            slab = pltpu.roll(v, _SEG_W - n, axis=1) if n else v
            u_ref[8 * a:8 * (a + 1), :] = slab

    x0 = 1920 - _TQ * i
    bias_t = u_ref[:, pl.ds(x0, _TK)]
    out_ref[0, 0] = attn_ref[0, 0] + bias_t
    bias_ref[0, 0] = bias_t


def kernel(attn_mtx, embedding_table):
    b, h, q, k = attn_mtx.shape
    assert (b, h, q, k) == (1, 16, 2048, 2048)
    seg = jnp.flip(embedding_table[2048:6143, :], axis=0).T
    seg = jnp.pad(seg, ((0, 0), (0, _SEG_W - seg.shape[1])))[:, None, :]

    grid = (h, q // _TQ)
    blk = pl.BlockSpec((1, 1, _TQ, _TK), lambda hh, ii: (0, hh, ii, 0))
    out, bias = pl.pallas_call(
        _rel_bias_kernel,
        grid=grid,
        in_specs=[
            pl.BlockSpec((1, 1, _SEG_W), lambda hh, ii: (hh, 0, 0)),
            blk,
        ],
        out_specs=[blk, blk],
        out_shape=[jax.ShapeDtypeStruct((b, h, q, k), jnp.float32)] * 2,
        scratch_shapes=[pltpu.VMEM((_TQ, _SEG_W), jnp.float32)],
    )(seg, attn_mtx)
    return out, bias


# speedup vs baseline: 59.9931x; 1.2383x over previous
import jax
import jax.numpy as jnp
from jax.experimental import pallas as pl
from jax.experimental.pallas import tpu as pltpu

_MAX_SEQLEN = 4096
_SEG_W = 4096
_TQ = 256
_TK = 2048


def _rel_bias_kernel(seg_ref, attn_ref, out_ref, bias_ref, u_ref):
    i = pl.program_id(1)

    @pl.when(i == 0)
    def _build_u():
        seg = seg_ref[0, 0, :]
        v = jnp.broadcast_to(seg[None, :], (8, _SEG_W))
        row = jax.lax.broadcasted_iota(jnp.int32, (8, _SEG_W), 0)
        for t in range(3):
            n = 1 << t
            rolled = pltpu.roll(v, _SEG_W - n, axis=1)
            v = jnp.where((row & n) == 0, rolled, v)
        for a in range(_TQ // 8):
            n = _TQ - 8 - 8 * a
            slab = pltpu.roll(v, _SEG_W - n, axis=1) if n else v
            u_ref[8 * a:8 * (a + 1), :] = slab

    x0 = (_MAX_SEQLEN // 2 - _TQ) - _TQ * i
    bias_t = u_ref[:, pl.ds(x0, _TK)]
    out_ref[0, 0] = attn_ref[0, 0] + bias_t
    bias_ref[0, 0] = bias_t


def kernel(attn_mtx, embedding_table):
    b, h, q, k = attn_mtx.shape
    assert (b, h, q, k) == (1, 16, 2048, 2048)
    seg = jnp.flip(embedding_table[2048:6143, :], axis=0).T
    seg = jnp.pad(seg, ((0, 0), (0, _SEG_W - seg.shape[1])))[:, None, :]

    grid = (h, q // _TQ)
    blk = pl.BlockSpec((1, 1, _TQ, _TK), lambda hh, ii: (0, hh, ii, 0))
    out, bias = pl.pallas_call(
        _rel_bias_kernel,
        grid=grid,
        in_specs=[
            pl.BlockSpec((1, 1, _SEG_W), lambda hh, ii: (hh, 0, 0)),
            blk,
        ],
        out_specs=[blk, blk],
        out_shape=[jax.ShapeDtypeStruct((b, h, q, k), jnp.float32)] * 2,
        scratch_shapes=[pltpu.VMEM((_TQ, _SEG_W), jnp.float32)],
        compiler_params=pltpu.CompilerParams(
            dimension_semantics=("parallel", "arbitrary"),
        ),
    )(seg, attn_mtx)
    return out, bias


# speedup vs baseline: 63.1442x; 1.0525x over previous
import jax
import jax.numpy as jnp
from jax.experimental import pallas as pl
from jax.experimental.pallas import tpu as pltpu

_MAX_SEQLEN = 4096
_SEG_W = 4096
_TQ = 512
_TK = 2048


def _rel_bias_kernel(seg_ref, attn_ref, out_ref, bias_ref, u_ref):
    i = pl.program_id(1)

    @pl.when(i == 0)
    def _build_u():
        seg = seg_ref[0, 0, :]
        v = jnp.broadcast_to(seg[None, :], (8, _SEG_W))
        row = jax.lax.broadcasted_iota(jnp.int32, (8, _SEG_W), 0)
        for t in range(3):
            n = 1 << t
            rolled = pltpu.roll(v, _SEG_W - n, axis=1)
            v = jnp.where((row & n) == 0, rolled, v)
        for a in range(_TQ // 8):
            n = _TQ - 8 - 8 * a
            slab = pltpu.roll(v, _SEG_W - n, axis=1) if n else v
            u_ref[8 * a:8 * (a + 1), :] = slab

    x0 = (_MAX_SEQLEN // 2 - _TQ) - _TQ * i
    bias_t = u_ref[:, pl.ds(x0, _TK)]
    out_ref[0, 0] = attn_ref[0, 0] + bias_t
    bias_ref[0, 0] = bias_t


def kernel(attn_mtx, embedding_table):
    b, h, q, k = attn_mtx.shape
    assert (b, h, q, k) == (1, 16, 2048, 2048)
    seg = jnp.flip(embedding_table[2048:6143, :], axis=0).T
    seg = jnp.pad(seg, ((0, 0), (0, _SEG_W - seg.shape[1])))[:, None, :]

    grid = (h, q // _TQ)
    blk = pl.BlockSpec((1, 1, _TQ, _TK), lambda hh, ii: (0, hh, ii, 0))
    out, bias = pl.pallas_call(
        _rel_bias_kernel,
        grid=grid,
        in_specs=[
            pl.BlockSpec((1, 1, _SEG_W), lambda hh, ii: (hh, 0, 0)),
            blk,
        ],
        out_specs=[blk, blk],
        out_shape=[jax.ShapeDtypeStruct((b, h, q, k), jnp.float32)] * 2,
        scratch_shapes=[pltpu.VMEM((_TQ, _SEG_W), jnp.float32)],
        compiler_params=pltpu.CompilerParams(
            dimension_semantics=("parallel", "arbitrary"),
        ),
    )(seg, attn_mtx)
    return out, bias


# speedup vs baseline: 63.4816x; 1.0053x over previous
import jax
import jax.numpy as jnp
from jax.experimental import pallas as pl
from jax.experimental.pallas import tpu as pltpu

_MAX_SEQLEN = 4096
_SEG_W = 4096
_TQ = 512
_TK = 2048


def _rel_bias_kernel(seg_ref, attn_ref, out_ref, bias_ref, u_ref):
    i = pl.program_id(1)

    @pl.when(i == 0)
    def _build_u():
        seg = seg_ref[0, 0, :]
        v = jnp.broadcast_to(seg[None, :], (8, _SEG_W))
        row = jax.lax.broadcasted_iota(jnp.int32, (8, _SEG_W), 0)
        for t in range(3):
            n = 1 << t
            rolled = pltpu.roll(v, _SEG_W - n, axis=1)
            v = jnp.where((row & n) == 0, rolled, v)
        for a in range(_TQ // 8):
            n = _TQ - 8 - 8 * a
            slab = pltpu.roll(v, _SEG_W - n, axis=1) if n else v
            u_ref[8 * a:8 * (a + 1), :] = slab

    for ii in range(2048 // _TQ):
        @pl.when(i == ii)
        def _consume(ii=ii):
            x0 = (_MAX_SEQLEN // 2 - _TQ) - _TQ * ii
            bias_t = u_ref[:, x0:x0 + _TK]
            out_ref[0, 0] = attn_ref[0, 0] + bias_t
            bias_ref[0, 0] = bias_t


def kernel(attn_mtx, embedding_table):
    b, h, q, k = attn_mtx.shape
    assert (b, h, q, k) == (1, 16, 2048, 2048)
    seg = jnp.flip(embedding_table[2048:6143, :], axis=0).T
    seg = jnp.pad(seg, ((0, 0), (0, _SEG_W - seg.shape[1])))[:, None, :]

    grid = (h, q // _TQ)
    blk = pl.BlockSpec((1, 1, _TQ, _TK), lambda hh, ii: (0, hh, ii, 0))
    out, bias = pl.pallas_call(
        _rel_bias_kernel,
        grid=grid,
        in_specs=[
            pl.BlockSpec((1, 1, _SEG_W), lambda hh, ii: (hh, 0, 0)),
            blk,
        ],
        out_specs=[blk, blk],
        out_shape=[jax.ShapeDtypeStruct((b, h, q, k), jnp.float32)] * 2,
        scratch_shapes=[pltpu.VMEM((_TQ, _SEG_W), jnp.float32)],
        compiler_params=pltpu.CompilerParams(
            dimension_semantics=("parallel", "arbitrary"),
        ),
    )(seg, attn_mtx)
    return out, bias
